# baseline (speedup 1.0000x reference)
import functools

import jax
import jax.numpy as jnp
from jax import lax
from jax.experimental import pallas as pl
from jax.experimental.pallas import tpu as pltpu
from jax.experimental.pallas import tpu_sc as plsc

B, S, C = 1024, 10, 25
V, D, H = 64, 400, 16
P = S * C
NC, NS, L = 2, 16, 16
NW = NC * NS
BLK = H * B


def _table_body(emb_tok_ref, w_ref, b_ref, posc_ref, out_ref):
    proj = jnp.dot(emb_tok_ref[...], w_ref[...],
                   preferred_element_type=jnp.float32)
    proj = proj + b_ref[...]
    out_ref[...] = posc_ref[...][:, None, :] + proj[None, :, :]


def _build_table(emb_tok, W, b, posc):
    return pl.pallas_call(
        _table_body,
        out_shape=jax.ShapeDtypeStruct((P, V, H), jnp.float32),
    )(emb_tok, W, b.reshape(1, H), posc)


def _sc_body(idxt_hbm, table_hbm, out_hbm, idx_v, tbl_v, rows_v, hsplat_v):
    wid = lax.axis_index("s") * NC + lax.axis_index("c")

    def p_body(i, carry):
        p = wid + i * NW
        s = p // C
        c = p % C
        pltpu.sync_copy(idxt_hbm.at[s, c], idx_v)
        pltpu.sync_copy(table_hbm.at[p], tbl_v)
        for b0 in range(0, B, L):
            iv = idx_v[pl.ds(b0, L)]
            base = (b0 // 128) * 1024 + b0 % 128
            for h in range(H):
                g = plsc.load_gather(tbl_v, [iv, hsplat_v[h, :]])
                rows_v[pl.ds((h // 8) * 8192 + (h % 8) * 128 + base, L)] = g
        off = pl.multiple_of(s * (50 * 8192) + c * BLK, 128)
        pltpu.sync_copy(rows_v, out_hbm.at[pl.ds(off, BLK)])
        return carry

    trips = jnp.where(wid < P % NW, P // NW + 1, P // NW)
    lax.fori_loop(0, trips, p_body, 0)


def _make_hsplat():
    return jnp.tile(jnp.arange(H, dtype=jnp.int32)[:, None], (1, L))


@functools.partial(
    pl.kernel,
    out_type=jax.ShapeDtypeStruct((S * 50 * 8192,), jnp.float32),
    mesh=plsc.VectorSubcoreMesh(core_axis_name="c", subcore_axis_name="s"),
    scratch_types=[
        pltpu.VMEM((B,), jnp.int32),
        pltpu.VMEM((V, H), jnp.float32),
        pltpu.VMEM((BLK,), jnp.float32),
    ],
    compiler_params=pltpu.CompilerParams(use_tc_tiling_on_sc=False,
                                         needs_layout_passes=False),
)
def _sc_lookup(idxt_hbm, table_hbm, hsplat_hbm, out_hbm, idx_v, tbl_v, rows_v):
    def run(hsplat_v):
        pltpu.sync_copy(hsplat_hbm, hsplat_v)
        _sc_body(idxt_hbm, table_hbm, out_hbm, idx_v, tbl_v, rows_v, hsplat_v)
    pl.run_scoped(run, pltpu.VMEM((H, L), jnp.int32))


def kernel(inputs, emb_tok, W, b, pos_emb):
    posc = pos_emb.reshape(P, H)
    combined = _build_table(emb_tok, W, b, posc)
    idxt = jnp.transpose(inputs, (1, 2, 0)).astype(jnp.int32)
    out_flat = _sc_lookup(idxt, combined, _make_hsplat())
    out5d = out_flat.reshape(S, 50, 8, 8, 128)
    return jnp.transpose(out5d, (2, 4, 0, 1, 3)).reshape(B, S, D)


# speedup vs baseline: 1.3201x; 1.3201x over previous
import functools

import jax
import jax.numpy as jnp
from jax import lax
from jax.experimental import pallas as pl
from jax.experimental.pallas import tpu as pltpu
from jax.experimental.pallas import tpu_sc as plsc

B, S, C = 1024, 10, 25
V, D, H = 64, 400, 16
P = S * C
NC, NS, L = 2, 16, 16
NW = NC * NS
BLK = H * B
MAXI = -(-P // NW)


def _table_body(emb_tok_ref, w_ref, b_ref, posc_ref, out_ref):
    proj = jnp.dot(emb_tok_ref[...], w_ref[...],
                   preferred_element_type=jnp.float32)
    proj = proj + b_ref[...]
    proj_t = proj.T
    out_ref[...] = posc_ref[...][:, :, None] + proj_t[None, :, :]


def _build_table(emb_tok, W, b, posc):
    return pl.pallas_call(
        _table_body,
        out_shape=jax.ShapeDtypeStruct((P, H, V), jnp.float32),
    )(emb_tok, W, b.reshape(1, H), posc)


def _compute_block(idx_ref, tbl_ref, rows_ref, hsplat_ref):
    for b0 in range(0, B, L):
        iv = idx_ref[pl.ds(b0, L)]
        base = (b0 // 128) * 1024 + b0 % 128
        for h in range(H):
            g = plsc.load_gather(tbl_ref, [hsplat_ref[h, :], iv])
            rows_ref[pl.ds((h // 8) * 8192 + (h % 8) * 128 + base, L)] = g


def _sc_body(idxt_hbm, table_hbm, out_hbm, idx_v, tbl_v, rows_v, hsplat_v,
             sem_idx, sem_tbl, sem_out):
    wid = lax.axis_index("s") * NC + lax.axis_index("c")

    def fire_inputs(i, k):
        p = wid + i * NW
        s = p // C
        c = p % C
        pltpu.async_copy(idxt_hbm.at[s, c], idx_v.at[k], sem_idx.at[k])
        pltpu.async_copy(table_hbm.at[p], tbl_v.at[k], sem_tbl.at[k])

    fire_inputs(0, 0)
    fire_inputs(1, 1)

    def pair_body(j, carry):
        for k in (0, 1):
            i = 2 * j + k
            p = wid + i * NW

            @pl.when(p < P)
            def _():
                pltpu.make_async_copy(
                    idxt_hbm.at[0, 0], idx_v.at[k], sem_idx.at[k]).wait()
                pltpu.make_async_copy(
                    table_hbm.at[0], tbl_v.at[k], sem_tbl.at[k]).wait()

                @pl.when(j >= 1)
                def _():
                    pltpu.make_async_copy(
                        out_hbm.at[pl.ds(0, BLK)], rows_v.at[k],
                        sem_out.at[k]).wait()

                _compute_block(idx_v.at[k], tbl_v.at[k], rows_v.at[k],
                               hsplat_v)
                s = p // C
                c = p % C
                off = pl.multiple_of(s * (50 * 8192) + c * BLK, 128)
                pltpu.async_copy(rows_v.at[k], out_hbm.at[pl.ds(off, BLK)],
                                 sem_out.at[k])

                @pl.when(p + 2 * NW < P)
                def _():
                    fire_inputs(i + 2, k)
        return carry

    lax.fori_loop(0, MAXI // 2, pair_body, 0)
    for k in (0, 1):
        pltpu.make_async_copy(
            out_hbm.at[pl.ds(0, BLK)], rows_v.at[k], sem_out.at[k]).wait()


def _make_hsplat():
    return jnp.tile(jnp.arange(H, dtype=jnp.int32)[:, None], (1, L))


@functools.partial(
    pl.kernel,
    out_type=jax.ShapeDtypeStruct((S * 50 * 8192,), jnp.float32),
    mesh=plsc.VectorSubcoreMesh(core_axis_name="c", subcore_axis_name="s"),
    scratch_types=[
        pltpu.VMEM((2, B), jnp.int32),
        pltpu.VMEM((2, H, V), jnp.float32),
        pltpu.VMEM((2, BLK), jnp.float32),
        pltpu.SemaphoreType.DMA((2,)),
        pltpu.SemaphoreType.DMA((2,)),
        pltpu.SemaphoreType.DMA((2,)),
    ],
    compiler_params=pltpu.CompilerParams(use_tc_tiling_on_sc=False,
                                         needs_layout_passes=False),
)
def _sc_lookup(idxt_hbm, table_hbm, hsplat_hbm, out_hbm,
               idx_v, tbl_v, rows_v, sem_idx, sem_tbl, sem_out):
    def run(hsplat_v):
        pltpu.sync_copy(hsplat_hbm, hsplat_v)
        _sc_body(idxt_hbm, table_hbm, out_hbm, idx_v, tbl_v, rows_v,
                 hsplat_v, sem_idx, sem_tbl, sem_out)
    pl.run_scoped(run, pltpu.VMEM((H, L), jnp.int32))


def kernel(inputs, emb_tok, W, b, pos_emb):
    posc = pos_emb.reshape(P, H)
    combined_t = _build_table(emb_tok, W, b, posc)
    idxt = jnp.transpose(inputs, (1, 2, 0)).astype(jnp.int32)
    out_flat = _sc_lookup(idxt, combined_t, _make_hsplat())
    out5d = out_flat.reshape(S, 50, 8, 8, 128)
    return jnp.transpose(out5d, (2, 4, 0, 1, 3)).reshape(B, S, D)
